# baseline (speedup 1.0000x reference)
import functools

import jax
import jax.numpy as jnp
from jax import lax
from jax.experimental import pallas as pl
from jax.experimental.pallas import tpu as pltpu
from jax.experimental.pallas import tpu_sc as plsc

VOCAB = 100000
DIM = 128
HOPS = 3
B = 1024
L = 50
M = 6

NC = 2
NS = 16
NW = NC * NS
R_TOT = HOPS * B * L
R_W = R_TOT // NW
CH_OUT = 32
CH_IDX = CH_OUT * M
G_SPLIT = 2
G_ROWS = CH_IDX // G_SPLIT
N_CH = R_W // CH_OUT
NBUF = 3
N_TRIP = N_CH // NBUF
LANES = 16


def _sc_gather_pool(c_flat, idx1d):
  mesh = plsc.VectorSubcoreMesh(core_axis_name="c", subcore_axis_name="s")

  scratch = [pltpu.VMEM((R_W * M,), jnp.int32)]
  scratch += [pltpu.VMEM((CH_IDX, DIM), jnp.float32) for _ in range(NBUF)]
  scratch += [pltpu.VMEM((CH_OUT, DIM), jnp.float32) for _ in range(NBUF)]
  scratch += [pltpu.SemaphoreType.DMA for _ in range(2 * NBUF)]

  @functools.partial(
      pl.kernel,
      mesh=mesh,
      out_type=jax.ShapeDtypeStruct((R_TOT, DIM), jnp.float32),
      scratch_types=scratch,
  )
  def k(c_hbm, idx_hbm, m_hbm, idx_v, *bufs):
    rows = bufs[:NBUF]
    outs = bufs[NBUF:2 * NBUF]
    gsems = bufs[2 * NBUF:3 * NBUF]
    ssems = bufs[3 * NBUF:4 * NBUF]
    wid = lax.axis_index("s") * NC + lax.axis_index("c")
    row0 = wid * R_W
    idx0 = wid * (R_W * M)

    pltpu.sync_copy(idx_hbm.at[pl.ds(idx0, R_W * M)], idx_v)

    def fire(c, t):
      for j in range(G_SPLIT):
        pltpu.async_copy(
            c_hbm.at[idx_v.at[pl.ds(c * CH_IDX + j * G_ROWS, G_ROWS)]],
            rows[t].at[pl.ds(j * G_ROWS, G_ROWS)],
            gsems[t])

    def drain(c, t):
      for j in range(G_SPLIT):
        pltpu.make_async_copy(
            c_hbm.at[idx_v.at[pl.ds(c * CH_IDX + j * G_ROWS, G_ROWS)]],
            rows[t].at[pl.ds(j * G_ROWS, G_ROWS)],
            gsems[t]).wait()

    def pool(t):
      def body(g2, inner):
        for h in range(2):
          g = g2 * 2 + h
          base = g * M
          for d in range(DIM // LANES):
            sl = pl.ds(d * LANES, LANES)
            acc = rows[t][base, sl]
            for j in range(1, M):
              acc = acc + rows[t][base + j, sl]
            outs[t][g, sl] = acc
        return inner

      lax.fori_loop(0, CH_OUT // 2, body, 0)

    def fire_store(c, t):
      pltpu.async_copy(
          outs[t], m_hbm.at[pl.ds(row0 + c * CH_OUT, CH_OUT)], ssems[t])

    def drain_store(c, t):
      pltpu.make_async_copy(
          outs[t], m_hbm.at[pl.ds(row0 + c * CH_OUT, CH_OUT)], ssems[t]).wait()

    for t in range(NBUF - 1):
      fire(t, t)

    def trip(k_, carry):
      c0 = k_ * NBUF
      for t in range(NBUF):
        c = c0 + t

        @pl.when(c + NBUF - 1 < N_CH)
        def _():
          fire(c + NBUF - 1, (t + NBUF - 1) % NBUF)

        drain(c, t)

        @pl.when(k_ > 0)
        def _():
          drain_store(c - NBUF, t)

        pool(t)
        fire_store(c, t)
      return carry

    lax.fori_loop(0, N_TRIP, trip, 0)
    for t in range(NBUF):
      drain_store(N_CH - NBUF + t, t)

  return k(c_flat, idx1d)


BB = 128


def _tc_body(m_ref, sig_ref, u_ref):
  m1 = m_ref[0]
  m2 = m_ref[1]
  m3 = m_ref[2]
  u = jnp.mean(m1, axis=1)
  for ma, mc in ((m1, m2), (m2, m3)):
    logits = jnp.sum(ma * u[:, None, :], axis=2)
    p = jax.nn.softmax(logits, axis=1)
    u = u + jnp.sum(mc * p[:, :, None], axis=1)
  sig_ref[...] = jax.nn.sigmoid(m3)
  u_ref[...] = u


def _tc_recurrence(m, interpret=False):
  return pl.pallas_call(
      _tc_body,
      grid=(B // BB,),
      in_specs=[pl.BlockSpec((HOPS, BB, L, DIM), lambda i: (0, i, 0, 0))],
      out_specs=[pl.BlockSpec((BB, L, DIM), lambda i: (i, 0, 0)),
                 pl.BlockSpec((BB, DIM), lambda i: (i, 0))],
      out_shape=[jax.ShapeDtypeStruct((B, L, DIM), jnp.float32),
                 jax.ShapeDtypeStruct((B, DIM), jnp.float32)],
      interpret=interpret,
  )(m)


def kernel(src_seqs, C):
  flat = src_seqs.reshape(-1).astype(jnp.int32)
  offs = (jnp.arange(1, HOPS + 1, dtype=jnp.int32) * VOCAB)[:, None]
  idx1d = (flat[None, :] + offs).reshape(-1)
  c_flat = C.reshape((HOPS + 1) * VOCAB, DIM)
  m = _sc_gather_pool(c_flat, idx1d)
  m = m.reshape(HOPS, B, L, DIM)
  sig, u = _tc_recurrence(m)
  return (sig, u[None])
